# baseline (speedup 1.0000x reference)
import functools

import jax
import jax.numpy as jnp
from jax import lax
from jax.experimental import pallas as pl
from jax.experimental.pallas import tpu as pltpu


def _rvq_kernel(x_ref, wp_ref, bp_ref, cb_ref, w1_ref, b1_ref, w2_ref,
                b2_ref, w3_ref, b3_ref,
                ang_ref, idx0_ref, idx1_ref, cl_ref, perp_ref,
                counts_scr, *, num_levels, num_codes, grid_b, grid_t):
    bi = pl.program_id(0)
    ti = pl.program_id(1)
    first = jnp.logical_and(bi == 0, ti == 0)
    last = jnp.logical_and(bi == grid_b - 1, ti == grid_t - 1)

    @pl.when(first)
    def _init():
        cl_ref[...] = jnp.zeros_like(cl_ref)
        perp_ref[...] = jnp.zeros_like(perp_ref)
        counts_scr[...] = jnp.zeros_like(counts_scr)

    xb = x_ref[0]
    r = lax.dot_general(xb, wp_ref[...], (((0,), (0,)), ((), ())),
                        preferred_element_type=jnp.float32)
    r = r + bp_ref[...]

    tt = r.shape[0]
    idx_refs = (idx0_ref, idx1_ref)
    q_total = jnp.zeros_like(r)
    loss_sum = jnp.zeros((), jnp.float32)
    counts = jnp.zeros((1, num_codes), jnp.float32)
    for lvl in range(num_levels):
        cb = cb_ref[lvl]
        rsq = jnp.sum(r * r, axis=1, keepdims=True)
        cbsq = jnp.sum(cb * cb, axis=1)[None, :]
        cross = lax.dot_general(r, cb, (((1,), (1,)), ((), ())),
                                preferred_element_type=jnp.float32)
        d = rsq - 2.0 * cross + cbsq
        idx = jnp.argmin(d, axis=1).astype(jnp.int32)
        onehot = (lax.broadcasted_iota(jnp.int32, (tt, num_codes), 1)
                  == idx[:, None]).astype(jnp.float32)
        q = lax.dot_general(onehot, cb, (((1,), (0,)), ((), ())),
                            preferred_element_type=jnp.float32,
                            precision=lax.Precision.HIGHEST)
        diff = r - q
        loss_sum = loss_sum + jnp.sum(diff * diff)
        counts = counts + jnp.sum(onehot, axis=0, keepdims=True)
        idx_refs[lvl][0, 0, :] = idx
        q_total = q_total + q
        r = diff

    h = jnp.maximum(lax.dot_general(q_total, w1_ref[...],
                                    (((1,), (0,)), ((), ())),
                                    preferred_element_type=jnp.float32)
                    + b1_ref[...], 0.0)
    h = jnp.maximum(lax.dot_general(h, w2_ref[...], (((1,), (0,)), ((), ())),
                                    preferred_element_type=jnp.float32)
                    + b2_ref[...], 0.0)
    dec = lax.dot_general(h, w3_ref[...], (((1,), (0,)), ((), ())),
                          preferred_element_type=jnp.float32) + b3_ref[...]
    ang_ref[0] = dec.T

    cl_ref[...] += loss_sum.reshape(1, 1)
    counts_scr[...] += counts

    @pl.when(last)
    def _finish():
        c = counts_scr[...]
        avg = c / jnp.sum(c)
        perp = jnp.exp(-jnp.sum(avg * jnp.log(avg + 1e-10)))
        perp_ref[...] = perp.reshape(1, 1)


def kernel(x, W_proj, b_proj, codebooks, W1, b1, W2, b2, W3, b3):
    B, C, T = x.shape
    L, K, D = codebooks.shape
    H = W1.shape[1]
    J = W3.shape[1]
    TT = min(512, T)
    grid_b, grid_t = B, T // TT
    N = B * T

    kern = functools.partial(_rvq_kernel, num_levels=L, num_codes=K,
                             grid_b=grid_b, grid_t=grid_t)

    full = lambda shape: pl.BlockSpec(shape, lambda b, t: (0,) * len(shape))
    out_shapes = (
        jax.ShapeDtypeStruct((B, J, T), jnp.float32),
        jax.ShapeDtypeStruct((B, 1, T), jnp.int32),
        jax.ShapeDtypeStruct((B, 1, T), jnp.int32),
        jax.ShapeDtypeStruct((1, 1), jnp.float32),
        jax.ShapeDtypeStruct((1, 1), jnp.float32),
    )
    out_specs = (
        pl.BlockSpec((1, J, TT), lambda b, t: (b, 0, t)),
        pl.BlockSpec((1, 1, TT), lambda b, t: (b, 0, t)),
        pl.BlockSpec((1, 1, TT), lambda b, t: (b, 0, t)),
        pl.BlockSpec((1, 1), lambda b, t: (0, 0)),
        pl.BlockSpec((1, 1), lambda b, t: (0, 0)),
    )
    in_specs = (
        pl.BlockSpec((1, C, TT), lambda b, t: (b, 0, t)),
        full((C, D)),
        full((1, D)),
        full((L, K, D)),
        full((D, H)),
        full((1, H)),
        full((H, H)),
        full((1, H)),
        full((H, J)),
        full((1, J)),
    )

    angles, idx0, idx1, cl_sum, perp = pl.pallas_call(
        kern,
        grid=(grid_b, grid_t),
        in_specs=in_specs,
        out_specs=out_specs,
        out_shape=out_shapes,
        scratch_shapes=[pltpu.VMEM((1, K), jnp.float32)],
        compiler_params=pltpu.CompilerParams(
            dimension_semantics=("arbitrary", "arbitrary")),
    )(x, W_proj, b_proj.reshape(1, D), codebooks, W1, b1.reshape(1, H),
      W2, b2.reshape(1, H), W3, b3.reshape(1, J))

    indices = jnp.concatenate(
        [idx0.reshape(1, N), idx1.reshape(1, N)], axis=0)
    cl = (cl_sum[0, 0] / (N * D)).astype(jnp.float32)
    codebook_loss = cl
    commit_loss = cl
    vq_loss = cl + 0.25 * cl
    perplexity = perp[0, 0]
    return (angles, indices, vq_loss, codebook_loss, commit_loss, perplexity)


# speedup vs baseline: 1.8769x; 1.8769x over previous
import functools

import jax
import jax.numpy as jnp
from jax import lax
from jax.experimental import pallas as pl
from jax.experimental.pallas import tpu as pltpu


def _rvq_kernel(x_ref, wp_ref, bp_ref, cb_ref, cbh_ref, cbl_ref,
                w1_ref, b1_ref, w2_ref, b2_ref, w3_ref, b3_ref,
                ang_ref, idx0_ref, idx1_ref, cl_ref, perp_ref,
                counts_scr, *, num_levels, num_codes, grid_b, grid_t):
    bi = pl.program_id(0)
    ti = pl.program_id(1)
    first = jnp.logical_and(bi == 0, ti == 0)
    last = jnp.logical_and(bi == grid_b - 1, ti == grid_t - 1)

    @pl.when(first)
    def _init():
        cl_ref[...] = jnp.zeros_like(cl_ref)
        perp_ref[...] = jnp.zeros_like(perp_ref)
        counts_scr[...] = jnp.zeros_like(counts_scr)

    xb = x_ref[0]
    r = lax.dot_general(xb, wp_ref[...], (((0,), (0,)), ((), ())),
                        preferred_element_type=jnp.float32)
    r = r + bp_ref[...]

    tt = r.shape[0]
    iota = lax.broadcasted_iota(jnp.int32, (tt, num_codes), 1)
    ones_row = jnp.ones((1, tt), jnp.bfloat16)
    idx_refs = (idx0_ref, idx1_ref)
    q_total = jnp.zeros_like(r)
    loss_sum = jnp.zeros((), jnp.float32)
    counts = jnp.zeros((1, num_codes), jnp.float32)
    for lvl in range(num_levels):
        cb = cb_ref[lvl]
        rsq = jnp.sum(r * r, axis=1, keepdims=True)
        cbsq = jnp.sum(cb * cb, axis=1)[None, :]
        cross = lax.dot_general(r, cb, (((1,), (1,)), ((), ())),
                                preferred_element_type=jnp.float32)
        d = rsq - 2.0 * cross + cbsq
        idx = jnp.argmin(d, axis=1).astype(jnp.int32)
        onehot = (iota == idx[:, None]).astype(jnp.float32).astype(jnp.bfloat16)
        q = (lax.dot_general(onehot, cbh_ref[lvl], (((1,), (0,)), ((), ())),
                             preferred_element_type=jnp.float32)
             + lax.dot_general(onehot, cbl_ref[lvl], (((1,), (0,)), ((), ())),
                               preferred_element_type=jnp.float32))
        diff = r - q
        loss_sum = loss_sum + jnp.sum(diff * diff)
        counts = counts + lax.dot_general(
            ones_row, onehot, (((1,), (0,)), ((), ())),
            preferred_element_type=jnp.float32)
        idx_refs[lvl][0, 0, :] = idx
        q_total = q_total + q
        r = diff

    h = jnp.maximum(lax.dot_general(q_total, w1_ref[...],
                                    (((1,), (0,)), ((), ())),
                                    preferred_element_type=jnp.float32)
                    + b1_ref[...], 0.0)
    h = jnp.maximum(lax.dot_general(h, w2_ref[...], (((1,), (0,)), ((), ())),
                                    preferred_element_type=jnp.float32)
                    + b2_ref[...], 0.0)
    dec = lax.dot_general(h, w3_ref[...], (((1,), (0,)), ((), ())),
                          preferred_element_type=jnp.float32) + b3_ref[...]
    ang_ref[0] = dec.T

    cl_ref[...] += loss_sum.reshape(1, 1)
    counts_scr[...] += counts

    @pl.when(last)
    def _finish():
        c = counts_scr[...]
        avg = c / jnp.sum(c)
        perp = jnp.exp(-jnp.sum(avg * jnp.log(avg + 1e-10)))
        perp_ref[...] = perp.reshape(1, 1)


def kernel(x, W_proj, b_proj, codebooks, W1, b1, W2, b2, W3, b3):
    B, C, T = x.shape
    L, K, D = codebooks.shape
    H = W1.shape[1]
    J = W3.shape[1]
    TT = min(1024, T)
    grid_b, grid_t = B, T // TT
    N = B * T

    cb_bits = lax.bitcast_convert_type(codebooks, jnp.uint32)
    cb_hi_f32 = lax.bitcast_convert_type(
        cb_bits & jnp.uint32(0xFFFF0000), jnp.float32)
    cb_hi = cb_hi_f32.astype(jnp.bfloat16)
    cb_lo = (codebooks - cb_hi_f32).astype(jnp.bfloat16)

    kern = functools.partial(_rvq_kernel, num_levels=L, num_codes=K,
                             grid_b=grid_b, grid_t=grid_t)

    full = lambda shape: pl.BlockSpec(shape, lambda b, t: (0,) * len(shape))
    out_shapes = (
        jax.ShapeDtypeStruct((B, J, T), jnp.float32),
        jax.ShapeDtypeStruct((B, 1, T), jnp.int32),
        jax.ShapeDtypeStruct((B, 1, T), jnp.int32),
        jax.ShapeDtypeStruct((1, 1), jnp.float32),
        jax.ShapeDtypeStruct((1, 1), jnp.float32),
    )
    out_specs = (
        pl.BlockSpec((1, J, TT), lambda b, t: (b, 0, t)),
        pl.BlockSpec((1, 1, TT), lambda b, t: (b, 0, t)),
        pl.BlockSpec((1, 1, TT), lambda b, t: (b, 0, t)),
        pl.BlockSpec((1, 1), lambda b, t: (0, 0)),
        pl.BlockSpec((1, 1), lambda b, t: (0, 0)),
    )
    in_specs = (
        pl.BlockSpec((1, C, TT), lambda b, t: (b, 0, t)),
        full((C, D)),
        full((1, D)),
        full((L, K, D)),
        full((L, K, D)),
        full((L, K, D)),
        full((D, H)),
        full((1, H)),
        full((H, H)),
        full((1, H)),
        full((H, J)),
        full((1, J)),
    )

    angles, idx0, idx1, cl_sum, perp = pl.pallas_call(
        kern,
        grid=(grid_b, grid_t),
        in_specs=in_specs,
        out_specs=out_specs,
        out_shape=out_shapes,
        scratch_shapes=[pltpu.VMEM((1, K), jnp.float32)],
        compiler_params=pltpu.CompilerParams(
            dimension_semantics=("arbitrary", "arbitrary")),
    )(x, W_proj, b_proj.reshape(1, D), codebooks, cb_hi, cb_lo,
      W1, b1.reshape(1, H), W2, b2.reshape(1, H), W3, b3.reshape(1, J))

    indices = jnp.concatenate(
        [idx0.reshape(1, N), idx1.reshape(1, N)], axis=0)
    cl = (cl_sum[0, 0] / (N * D)).astype(jnp.float32)
    codebook_loss = cl
    commit_loss = cl
    vq_loss = cl + 0.25 * cl
    perplexity = perp[0, 0]
    return (angles, indices, vq_loss, codebook_loss, commit_loss, perplexity)
